# Initial kernel scaffold; baseline (speedup 1.0000x reference)
import jax
import jax.numpy as jnp
from jax.experimental import pallas as pl


def kernel(inputs, structure, W_ih, b_ih, W_hh, b_hh, Wa, ba, W1, b1, W2, Wv):
    raise NotImplementedError("write your pallas kernel here")


# speedup vs baseline: 14.7668x; 14.7668x over previous
import jax
import jax.numpy as jnp
from jax import lax
from jax.experimental import pallas as pl
from jax.experimental.pallas import tpu as pltpu

_B = 8
_IN = 128
_H = 800
_C = 800
_A = 128
_HEADS = 8
_DH = _C // _HEADS
_RADIUS2 = 64.0
_TOTAL = 256
_L = _TOTAL

_PREC = lax.Precision.HIGHEST


def _dot(a, b):
    return jnp.dot(a, b, precision=_PREC, preferred_element_type=jnp.float32)


def _rnn_kernel(lmax_ref, data_ref,
                wih_r_ref, wih_z_ref, wih_n_ref, bih_ref,
                whh_r_ref, whh_z_ref, whh_n_ref, bhh_ref,
                wa_ref, ba_ref, w1f_ref, b1_ref, w1n_ref,
                w2_ref, mw_ref, wvf_ref, wvn_ref,
                poss_ref, angs_ref, v_ref, h1_ref):
    f32 = jnp.float32
    poss_ref[...] = jnp.zeros((_L, _B, 9), f32)
    angs_ref[...] = jnp.zeros((_L, _B, 3), f32)
    v_ref[...] = jnp.zeros((_L, _B, _C), f32)
    h1_ref[...] = jnp.zeros((_L, _B, _A), f32)

    e_mat = (lax.broadcasted_iota(jnp.int32, (_HEADS, _C), 1) // _DH ==
             lax.broadcasted_iota(jnp.int32, (_HEADS, _C), 0)).astype(f32)
    p_mat = (lax.broadcasted_iota(jnp.int32, (_B, _L * _B), 1) % _B ==
             lax.broadcasted_iota(jnp.int32, (_B, _L * _B), 0)).astype(f32)
    jidx = lax.broadcasted_iota(jnp.int32, (_L, _B, 1), 0)

    bih = bih_ref[...]
    bhh = bhh_ref[...]
    w2 = w2_ref[...]
    mw = mw_ref[...]
    b1 = b1_ref[...]
    ba = ba_ref[...]

    def body(t, carry):
        h, ctx = carry
        tm1 = jnp.maximum(t - 1, 0)
        prev_row = poss_ref[tm1]
        is0 = (t == 0)
        base = jnp.where(is0, jnp.zeros((_B, 3), f32), prev_row[:, 6:9])

        dt = data_ref[t]
        x = jnp.concatenate([dt, ctx], axis=1)

        gi_r = _dot(x, wih_r_ref[...]) + bih[:, 0:_H]
        gi_z = _dot(x, wih_z_ref[...]) + bih[:, _H:2 * _H]
        gi_n = _dot(x, wih_n_ref[...]) + bih[:, 2 * _H:3 * _H]
        gh_r = _dot(h, whh_r_ref[...]) + bhh[:, 0:_H]
        gh_z = _dot(h, whh_z_ref[...]) + bhh[:, _H:2 * _H]
        gh_n = _dot(h, whh_n_ref[...]) + bhh[:, 2 * _H:3 * _H]
        r = jax.nn.sigmoid(gi_r + gh_r)
        z = jax.nn.sigmoid(gi_z + gh_z)
        n = jnp.tanh(gi_n + r * gh_n)
        h_new = (1.0 - z) * n + z * h

        angle = jnp.pi * jnp.tanh(_dot(h_new, wa_ref[...]) + ba)

        cosv = jnp.cos(angle)
        sinv = jnp.sin(angle)
        rn = lax.rsqrt(cosv * cosv + sinv * sinv + 0.01)
        ux = 1.5 * cosv * rn
        uy = 1.5 * sinv * rn
        uz = 0.15 * rn
        a0x = base[:, 0:1] + ux[:, 0:1]
        a0y = base[:, 1:2] + uy[:, 0:1]
        a0z = base[:, 2:3] + uz[:, 0:1]
        a1x = a0x + ux[:, 1:2]
        a1y = a0y + uy[:, 1:2]
        a1z = a0z + uz[:, 1:2]
        a2x = a1x + ux[:, 2:3]
        a2y = a1y + uy[:, 2:3]
        a2z = a1z + uz[:, 2:3]
        prow = jnp.concatenate(
            [a0x, a0y, a0z, a1x, a1y, a1z, a2x, a2y, a2z], axis=1)

        poss_ref[t] = prow
        angs_ref[t] = angle
        frow = jnp.concatenate([h_new, dt, angle], axis=1)
        v_ref[t] = _dot(frow, wvf_ref[...])
        h1_ref[t] = _dot(frow, w1f_ref[...]) + b1

        possv = poss_ref[...]
        dsq = possv - prev_row[None, :, :]
        dsq = dsq * dsq
        s3 = dsq[:, :, 0:3] + dsq[:, :, 3:6] + dsq[:, :, 6:9]
        ndist = jnp.sqrt(s3).reshape(_L * _B, 3)
        d2 = jnp.sum(dsq[:, :, 3:6], axis=2, keepdims=True)
        maskf = ((jidx < t) & (d2 <= _RADIUS2)).astype(f32).reshape(_L * _B, 1)

        h1v = h1_ref[...].reshape(_L * _B, _A)
        hid = jnp.tanh(h1v + _dot(ndist, w1n_ref[...]))
        logits = _dot(hid, w2) - mw
        w = jnp.exp(logits) * maskf
        den = _dot(p_mat, w)
        wfull = _dot(w, e_mat)
        vals = v_ref[...].reshape(_L * _B, _C) + _dot(ndist, wvn_ref[...])
        num = _dot(p_mat, wfull * vals)
        den_full = _dot(den, e_mat)
        ctx_new = num / jnp.maximum(den_full, 1e-30)
        ctx_new = jnp.where(is0, jnp.zeros((_B, _C), f32), ctx_new)
        return h_new, ctx_new

    h0 = jnp.zeros((_B, _H), f32)
    c0 = jnp.zeros((_B, _C), f32)
    lax.fori_loop(0, lmax_ref[0], body, (h0, c0))


def kernel(inputs, structure, W_ih, b_ih, W_hh, b_hh, Wa, ba, W1, b1, W2, Wv):
    f32 = jnp.float32
    counts = jnp.bincount(structure, length=_B)
    starts = jnp.concatenate(
        [jnp.zeros((1,), counts.dtype), jnp.cumsum(counts)[:-1]])
    pos = jnp.arange(_TOTAL) - starts[structure]
    data = jnp.zeros((_L, _B, _IN), f32).at[pos, structure].set(inputs)
    lmax = jnp.max(counts).astype(jnp.int32).reshape(1)

    WihT = W_ih.T.astype(f32)
    WhhT = W_hh.T.astype(f32)
    mw = jnp.sum(jnp.abs(W2), axis=0).reshape(1, _HEADS)

    poss_out, angs_out = pl.pallas_call(
        _rnn_kernel,
        in_specs=[pl.BlockSpec(memory_space=pltpu.SMEM)] +
                 [pl.BlockSpec(memory_space=pltpu.VMEM)] * 18,
        out_specs=[pl.BlockSpec(memory_space=pltpu.VMEM)] * 2,
        out_shape=[
            jax.ShapeDtypeStruct((_L, _B, 9), f32),
            jax.ShapeDtypeStruct((_L, _B, 3), f32),
        ],
        scratch_shapes=[
            pltpu.VMEM((_L, _B, _C), f32),
            pltpu.VMEM((_L, _B, _A), f32),
        ],
    )(
        lmax, data,
        WihT[:, 0:_H], WihT[:, _H:2 * _H], WihT[:, 2 * _H:3 * _H],
        b_ih.reshape(1, 3 * _H),
        WhhT[:, 0:_H], WhhT[:, _H:2 * _H], WhhT[:, 2 * _H:3 * _H],
        b_hh.reshape(1, 3 * _H),
        Wa, ba.reshape(1, 3),
        W1[0:_H + _IN + 3], b1.reshape(1, _A), W1[_H + _IN + 3:],
        W2, mw, Wv[0:_H + _IN + 3], Wv[_H + _IN + 3:],
    )

    pos_flat = poss_out[pos, structure].reshape(_TOTAL, 3, 3)
    ang_flat = angs_out[pos, structure]
    return pos_flat, ang_flat


# speedup vs baseline: 60.1060x; 4.0703x over previous
import jax
import jax.numpy as jnp
from jax import lax
from jax.experimental import pallas as pl
from jax.experimental.pallas import tpu as pltpu

_B = 8
_IN = 128
_H = 800
_C = 800
_A = 128
_HEADS = 8
_DH = _C // _HEADS
_RADIUS2 = 64.0
_TOTAL = 256
_L = _TOTAL

_PREC = lax.Precision.DEFAULT


def _dot(a, b):
    return jnp.dot(a, b, precision=_PREC, preferred_element_type=jnp.float32)


def _rnn_kernel(lmax_ref, data_ref,
                wih_r_ref, wih_z_ref, wih_n_ref, bih_ref,
                whh_r_ref, whh_z_ref, whh_n_ref, bhh_ref,
                wa_ref, ba_ref, w1f_ref, b1_ref, w1n_ref,
                w2_ref, mw_ref, wvf_ref, wvn_ref,
                poss_ref, angs_ref, v_ref, h1_ref):
    f32 = jnp.float32
    poss_ref[...] = jnp.zeros((_L, _B, 9), f32)
    angs_ref[...] = jnp.zeros((_L, _B, 3), f32)
    v_ref[...] = jnp.zeros((_L, _B, _C), f32)
    h1_ref[...] = jnp.zeros((_L, _B, _A), f32)

    e_mat = (lax.broadcasted_iota(jnp.int32, (_HEADS, _C), 1) // _DH ==
             lax.broadcasted_iota(jnp.int32, (_HEADS, _C), 0)).astype(f32)
    p_mat = (lax.broadcasted_iota(jnp.int32, (_B, _L * _B), 1) % _B ==
             lax.broadcasted_iota(jnp.int32, (_B, _L * _B), 0)).astype(f32)
    jidx = lax.broadcasted_iota(jnp.int32, (_L, _B, 1), 0)

    bih = bih_ref[...]
    bhh = bhh_ref[...]
    w2 = w2_ref[...]
    mw = mw_ref[...]
    b1 = b1_ref[...]
    ba = ba_ref[...]

    def body(t, carry):
        h, ctx = carry
        tm1 = jnp.maximum(t - 1, 0)
        prev_row = poss_ref[tm1]
        is0 = (t == 0)
        base = jnp.where(is0, jnp.zeros((_B, 3), f32), prev_row[:, 6:9])

        dt = data_ref[t]
        x = jnp.concatenate([dt, ctx], axis=1)

        gi_r = _dot(x, wih_r_ref[...]) + bih[:, 0:_H]
        gi_z = _dot(x, wih_z_ref[...]) + bih[:, _H:2 * _H]
        gi_n = _dot(x, wih_n_ref[...]) + bih[:, 2 * _H:3 * _H]
        gh_r = _dot(h, whh_r_ref[...]) + bhh[:, 0:_H]
        gh_z = _dot(h, whh_z_ref[...]) + bhh[:, _H:2 * _H]
        gh_n = _dot(h, whh_n_ref[...]) + bhh[:, 2 * _H:3 * _H]
        r = jax.nn.sigmoid(gi_r + gh_r)
        z = jax.nn.sigmoid(gi_z + gh_z)
        n = jnp.tanh(gi_n + r * gh_n)
        h_new = (1.0 - z) * n + z * h

        angle = jnp.pi * jnp.tanh(_dot(h_new, wa_ref[...]) + ba)

        cosv = jnp.cos(angle)
        sinv = jnp.sin(angle)
        rn = lax.rsqrt(cosv * cosv + sinv * sinv + 0.01)
        ux = 1.5 * cosv * rn
        uy = 1.5 * sinv * rn
        uz = 0.15 * rn
        a0x = base[:, 0:1] + ux[:, 0:1]
        a0y = base[:, 1:2] + uy[:, 0:1]
        a0z = base[:, 2:3] + uz[:, 0:1]
        a1x = a0x + ux[:, 1:2]
        a1y = a0y + uy[:, 1:2]
        a1z = a0z + uz[:, 1:2]
        a2x = a1x + ux[:, 2:3]
        a2y = a1y + uy[:, 2:3]
        a2z = a1z + uz[:, 2:3]
        prow = jnp.concatenate(
            [a0x, a0y, a0z, a1x, a1y, a1z, a2x, a2y, a2z], axis=1)

        poss_ref[t] = prow
        angs_ref[t] = angle
        frow = jnp.concatenate([h_new, dt, angle], axis=1)
        v_ref[t] = _dot(frow, wvf_ref[...])
        h1_ref[t] = _dot(frow, w1f_ref[...]) + b1

        possv = poss_ref[...]
        dsq = possv - prev_row[None, :, :]
        dsq = dsq * dsq
        s3 = dsq[:, :, 0:3] + dsq[:, :, 3:6] + dsq[:, :, 6:9]
        ndist = jnp.sqrt(s3).reshape(_L * _B, 3)
        d2 = jnp.sum(dsq[:, :, 3:6], axis=2, keepdims=True)
        maskf = ((jidx < t) & (d2 <= _RADIUS2)).astype(f32).reshape(_L * _B, 1)

        h1v = h1_ref[...].reshape(_L * _B, _A)
        hid = jnp.tanh(h1v + _dot(ndist, w1n_ref[...]))
        logits = _dot(hid, w2) - mw
        w = jnp.exp(logits) * maskf
        den = _dot(p_mat, w)
        wfull = _dot(w, e_mat)
        vals = v_ref[...].reshape(_L * _B, _C) + _dot(ndist, wvn_ref[...])
        num = _dot(p_mat, wfull * vals)
        den_full = _dot(den, e_mat)
        ctx_new = num / jnp.maximum(den_full, 1e-30)
        ctx_new = jnp.where(is0, jnp.zeros((_B, _C), f32), ctx_new)
        return h_new, ctx_new

    h0 = jnp.zeros((_B, _H), f32)
    c0 = jnp.zeros((_B, _C), f32)
    lax.fori_loop(0, lmax_ref[0], body, (h0, c0))


def kernel(inputs, structure, W_ih, b_ih, W_hh, b_hh, Wa, ba, W1, b1, W2, Wv):
    f32 = jnp.float32
    counts = jnp.bincount(structure, length=_B)
    starts = jnp.concatenate(
        [jnp.zeros((1,), counts.dtype), jnp.cumsum(counts)[:-1]])
    pos = jnp.arange(_TOTAL) - starts[structure]
    data = jnp.zeros((_L, _B, _IN), f32).at[pos, structure].set(inputs)
    lmax = jnp.max(counts).astype(jnp.int32).reshape(1)

    WihT = W_ih.T.astype(f32)
    WhhT = W_hh.T.astype(f32)
    mw = jnp.sum(jnp.abs(W2), axis=0).reshape(1, _HEADS)

    poss_out, angs_out = pl.pallas_call(
        _rnn_kernel,
        in_specs=[pl.BlockSpec(memory_space=pltpu.SMEM)] +
                 [pl.BlockSpec(memory_space=pltpu.VMEM)] * 18,
        out_specs=[pl.BlockSpec(memory_space=pltpu.VMEM)] * 2,
        out_shape=[
            jax.ShapeDtypeStruct((_L, _B, 9), f32),
            jax.ShapeDtypeStruct((_L, _B, 3), f32),
        ],
        scratch_shapes=[
            pltpu.VMEM((_L, _B, _C), f32),
            pltpu.VMEM((_L, _B, _A), f32),
        ],
    )(
        lmax, data,
        WihT[:, 0:_H], WihT[:, _H:2 * _H], WihT[:, 2 * _H:3 * _H],
        b_ih.reshape(1, 3 * _H),
        WhhT[:, 0:_H], WhhT[:, _H:2 * _H], WhhT[:, 2 * _H:3 * _H],
        b_hh.reshape(1, 3 * _H),
        Wa, ba.reshape(1, 3),
        W1[0:_H + _IN + 3], b1.reshape(1, _A), W1[_H + _IN + 3:],
        W2, mw, Wv[0:_H + _IN + 3], Wv[_H + _IN + 3:],
    )

    pos_flat = poss_out[pos, structure].reshape(_TOTAL, 3, 3)
    ang_flat = angs_out[pos, structure]
    return pos_flat, ang_flat


# speedup vs baseline: 83.1122x; 1.3828x over previous
import jax
import jax.numpy as jnp
from jax import lax
from jax.experimental import pallas as pl
from jax.experimental.pallas import tpu as pltpu

_B = 8
_IN = 128
_H = 800
_C = 800
_A = 128
_HEADS = 8
_DH = _C // _HEADS
_RADIUS2 = 64.0
_TOTAL = 256
_L = _TOTAL

_PREC = lax.Precision.DEFAULT
_JB = 64


def _dot(a, b):
    return jnp.dot(a, b, precision=_PREC, preferred_element_type=jnp.float32)


def _rnn_kernel(lmax_ref, data_ref,
                wih_r_ref, wih_z_ref, wih_n_ref, bih_ref,
                whh_r_ref, whh_z_ref, whh_n_ref, bhh_ref,
                wa_ref, ba_ref, w1f_ref, b1_ref, w1n_ref,
                w2_ref, mw_ref, wvf_ref, wvn_ref,
                poss_ref, angs_ref, v_ref, h1_ref):
    f32 = jnp.float32
    poss_ref[...] = jnp.zeros((_L, _B, 9), f32)
    angs_ref[...] = jnp.zeros((_L, _B, 3), f32)
    v_ref[...] = jnp.zeros((_L, _B, _C), f32)
    h1_ref[...] = jnp.zeros((_L, _B, _A), f32)

    e_mat = (lax.broadcasted_iota(jnp.int32, (_HEADS, _C), 1) // _DH ==
             lax.broadcasted_iota(jnp.int32, (_HEADS, _C), 0)).astype(f32)
    p_mat = (lax.broadcasted_iota(jnp.int32, (_B, _JB * _B), 1) % _B ==
             lax.broadcasted_iota(jnp.int32, (_B, _JB * _B), 0)).astype(f32)
    jidx = lax.broadcasted_iota(jnp.int32, (_JB, _B, 1), 0)

    bih = bih_ref[...]
    bhh = bhh_ref[...]
    w2 = w2_ref[...]
    mw = mw_ref[...]
    b1 = b1_ref[...]
    ba = ba_ref[...]

    def body(t, carry):
        h, ctx = carry
        tm1 = jnp.maximum(t - 1, 0)
        prev_row = poss_ref[tm1]
        is0 = (t == 0)
        base = jnp.where(is0, jnp.zeros((_B, 3), f32), prev_row[:, 6:9])

        dt = data_ref[t]
        x = jnp.concatenate([dt, ctx], axis=1)

        gi_r = _dot(x, wih_r_ref[...]) + bih[:, 0:_H]
        gi_z = _dot(x, wih_z_ref[...]) + bih[:, _H:2 * _H]
        gi_n = _dot(x, wih_n_ref[...]) + bih[:, 2 * _H:3 * _H]
        gh_r = _dot(h, whh_r_ref[...]) + bhh[:, 0:_H]
        gh_z = _dot(h, whh_z_ref[...]) + bhh[:, _H:2 * _H]
        gh_n = _dot(h, whh_n_ref[...]) + bhh[:, 2 * _H:3 * _H]
        r = jax.nn.sigmoid(gi_r + gh_r)
        z = jax.nn.sigmoid(gi_z + gh_z)
        n = jnp.tanh(gi_n + r * gh_n)
        h_new = (1.0 - z) * n + z * h

        angle = jnp.pi * jnp.tanh(_dot(h_new, wa_ref[...]) + ba)

        cosv = jnp.cos(angle)
        sinv = jnp.sin(angle)
        rn = lax.rsqrt(cosv * cosv + sinv * sinv + 0.01)
        ux = 1.5 * cosv * rn
        uy = 1.5 * sinv * rn
        uz = 0.15 * rn
        a0x = base[:, 0:1] + ux[:, 0:1]
        a0y = base[:, 1:2] + uy[:, 0:1]
        a0z = base[:, 2:3] + uz[:, 0:1]
        a1x = a0x + ux[:, 1:2]
        a1y = a0y + uy[:, 1:2]
        a1z = a0z + uz[:, 1:2]
        a2x = a1x + ux[:, 2:3]
        a2y = a1y + uy[:, 2:3]
        a2z = a1z + uz[:, 2:3]
        prow = jnp.concatenate(
            [a0x, a0y, a0z, a1x, a1y, a1z, a2x, a2y, a2z], axis=1)

        poss_ref[t] = prow
        angs_ref[t] = angle
        frow = jnp.concatenate([h_new, dt, angle], axis=1)
        v_ref[t] = _dot(frow, wvf_ref[...])
        h1_ref[t] = _dot(frow, w1f_ref[...]) + b1

        def blk(i, acc):
            num_a, den_a = acc
            possv = poss_ref[pl.ds(i * _JB, _JB)]
            dsq = possv - prev_row[None, :, :]
            dsq = dsq * dsq
            s3 = dsq[:, :, 0:3] + dsq[:, :, 3:6] + dsq[:, :, 6:9]
            ndist = jnp.sqrt(s3).reshape(_JB * _B, 3)
            d2 = jnp.sum(dsq[:, :, 3:6], axis=2, keepdims=True)
            maskf = ((i * _JB + jidx < t) &
                     (d2 <= _RADIUS2)).astype(f32).reshape(_JB * _B, 1)

            h1v = h1_ref[pl.ds(i * _JB, _JB)].reshape(_JB * _B, _A)
            hid = jnp.tanh(h1v + _dot(ndist, w1n_ref[...]))
            logits = _dot(hid, w2) - mw
            w = jnp.exp(logits) * maskf
            den_a = den_a + _dot(p_mat, w)
            wfull = _dot(w, e_mat)
            vals = (v_ref[pl.ds(i * _JB, _JB)].reshape(_JB * _B, _C) +
                    _dot(ndist, wvn_ref[...]))
            num_a = num_a + _dot(p_mat, wfull * vals)
            return num_a, den_a

        nblk = (t + _JB - 1) // _JB
        num, den = lax.fori_loop(
            0, nblk, blk,
            (jnp.zeros((_B, _C), f32), jnp.zeros((_B, _HEADS), f32)))
        den_full = _dot(den, e_mat)
        ctx_new = num / jnp.maximum(den_full, 1e-30)
        ctx_new = jnp.where(is0, jnp.zeros((_B, _C), f32), ctx_new)
        return h_new, ctx_new

    h0 = jnp.zeros((_B, _H), f32)
    c0 = jnp.zeros((_B, _C), f32)
    lax.fori_loop(0, lmax_ref[0], body, (h0, c0))


def kernel(inputs, structure, W_ih, b_ih, W_hh, b_hh, Wa, ba, W1, b1, W2, Wv):
    f32 = jnp.float32
    counts = jnp.bincount(structure, length=_B)
    starts = jnp.concatenate(
        [jnp.zeros((1,), counts.dtype), jnp.cumsum(counts)[:-1]])
    pos = jnp.arange(_TOTAL) - starts[structure]
    data = jnp.zeros((_L, _B, _IN), f32).at[pos, structure].set(inputs)
    lmax = jnp.max(counts).astype(jnp.int32).reshape(1)

    WihT = W_ih.T.astype(f32)
    WhhT = W_hh.T.astype(f32)
    mw = jnp.sum(jnp.abs(W2), axis=0).reshape(1, _HEADS)

    poss_out, angs_out = pl.pallas_call(
        _rnn_kernel,
        in_specs=[pl.BlockSpec(memory_space=pltpu.SMEM)] +
                 [pl.BlockSpec(memory_space=pltpu.VMEM)] * 18,
        out_specs=[pl.BlockSpec(memory_space=pltpu.VMEM)] * 2,
        out_shape=[
            jax.ShapeDtypeStruct((_L, _B, 9), f32),
            jax.ShapeDtypeStruct((_L, _B, 3), f32),
        ],
        scratch_shapes=[
            pltpu.VMEM((_L, _B, _C), f32),
            pltpu.VMEM((_L, _B, _A), f32),
        ],
    )(
        lmax, data,
        WihT[:, 0:_H], WihT[:, _H:2 * _H], WihT[:, 2 * _H:3 * _H],
        b_ih.reshape(1, 3 * _H),
        WhhT[:, 0:_H], WhhT[:, _H:2 * _H], WhhT[:, 2 * _H:3 * _H],
        b_hh.reshape(1, 3 * _H),
        Wa, ba.reshape(1, 3),
        W1[0:_H + _IN + 3], b1.reshape(1, _A), W1[_H + _IN + 3:],
        W2, mw, Wv[0:_H + _IN + 3], Wv[_H + _IN + 3:],
    )

    pos_flat = poss_out[pos, structure].reshape(_TOTAL, 3, 3)
    ang_flat = angs_out[pos, structure]
    return pos_flat, ang_flat


# speedup vs baseline: 84.8576x; 1.0210x over previous
import jax
import jax.numpy as jnp
from jax import lax
from jax.experimental import pallas as pl
from jax.experimental.pallas import tpu as pltpu

_B = 8
_IN = 128
_H = 800
_C = 800
_A = 128
_HEADS = 8
_DH = _C // _HEADS
_RADIUS2 = 64.0
_TOTAL = 256
_L = _TOTAL

_PREC = lax.Precision.DEFAULT
_JB = 64


def _dot(a, b):
    return jnp.dot(a, b, precision=_PREC, preferred_element_type=jnp.float32)


def _rnn_kernel(lmax_ref, data_ref,
                wih_r_ref, wih_z_ref, wih_n_ref, bih_ref,
                whh_r_ref, whh_z_ref, whh_n_ref, bhh_ref,
                wa_ref, ba_ref, w1f_ref, b1_ref, w1n_ref,
                w2_ref, mw_ref, wvf_ref, wvn_ref,
                poss_ref, angs_ref, v_ref, h1_ref):
    f32 = jnp.float32
    poss_ref[...] = jnp.zeros((_L, _B, 9), f32)
    angs_ref[...] = jnp.zeros((_L, _B, 3), f32)
    v_ref[...] = jnp.zeros((_L, _B, _C), f32)
    h1_ref[...] = jnp.zeros((_L, _B, _A), f32)

    e_mat = (lax.broadcasted_iota(jnp.int32, (_HEADS, _C), 1) // _DH ==
             lax.broadcasted_iota(jnp.int32, (_HEADS, _C), 0)).astype(f32)
    p_mat = (lax.broadcasted_iota(jnp.int32, (_B, _JB * _B), 1) % _B ==
             lax.broadcasted_iota(jnp.int32, (_B, _JB * _B), 0)).astype(f32)
    jidx = lax.broadcasted_iota(jnp.int32, (_JB, _B, 1), 0)

    bih = bih_ref[...]
    bhh = bhh_ref[...]
    w2 = w2_ref[...]
    mw = mw_ref[...]
    b1 = b1_ref[...]
    ba = ba_ref[...]

    def body(t, carry):
        h, ctx = carry
        tm1 = jnp.maximum(t - 1, 0)
        prev_row = poss_ref[tm1]
        is0 = (t == 0)
        base = jnp.where(is0, jnp.zeros((_B, 3), f32), prev_row[:, 6:9])

        def blk(i, acc):
            num_a, den_a = acc
            possv = poss_ref[pl.ds(i * _JB, _JB)]
            dsq = possv - prev_row[None, :, :]
            dsq = dsq * dsq
            s3 = dsq[:, :, 0:3] + dsq[:, :, 3:6] + dsq[:, :, 6:9]
            ndist = jnp.sqrt(s3).reshape(_JB * _B, 3)
            d2 = jnp.sum(dsq[:, :, 3:6], axis=2, keepdims=True)
            maskf = ((i * _JB + jidx < t) &
                     (d2 <= _RADIUS2)).astype(f32).reshape(_JB * _B, 1)

            h1v = h1_ref[pl.ds(i * _JB, _JB)].reshape(_JB * _B, _A)
            hid = jnp.tanh(h1v + _dot(ndist, w1n_ref[...]))
            logits = _dot(hid, w2) - mw
            w = jnp.exp(logits) * maskf
            den_a = den_a + _dot(p_mat, w)
            wfull = _dot(w, e_mat)
            vals = (v_ref[pl.ds(i * _JB, _JB)].reshape(_JB * _B, _C) +
                    _dot(ndist, wvn_ref[...]))
            num_a = num_a + _dot(p_mat, wfull * vals)
            return num_a, den_a

        nblk = (t + _JB - 1) // _JB
        num, den = lax.fori_loop(
            0, nblk, blk,
            (jnp.zeros((_B, _C), f32), jnp.zeros((_B, _HEADS), f32)))
        den_full = _dot(den, e_mat)
        ctx_new = num / jnp.maximum(den_full, 1e-30)
        ctx_new = jnp.where(is0, jnp.zeros((_B, _C), f32), ctx_new)

        dt = data_ref[t]
        x = jnp.concatenate([dt, ctx], axis=1)

        gi_r = _dot(x, wih_r_ref[...]) + bih[:, 0:_H]
        gi_z = _dot(x, wih_z_ref[...]) + bih[:, _H:2 * _H]
        gi_n = _dot(x, wih_n_ref[...]) + bih[:, 2 * _H:3 * _H]
        gh_r = _dot(h, whh_r_ref[...]) + bhh[:, 0:_H]
        gh_z = _dot(h, whh_z_ref[...]) + bhh[:, _H:2 * _H]
        gh_n = _dot(h, whh_n_ref[...]) + bhh[:, 2 * _H:3 * _H]
        r = jax.nn.sigmoid(gi_r + gh_r)
        z = jax.nn.sigmoid(gi_z + gh_z)
        n = jnp.tanh(gi_n + r * gh_n)
        h_new = (1.0 - z) * n + z * h

        angle = jnp.pi * jnp.tanh(_dot(h_new, wa_ref[...]) + ba)

        cosv = jnp.cos(angle)
        sinv = jnp.sin(angle)
        rn = lax.rsqrt(cosv * cosv + sinv * sinv + 0.01)
        ux = 1.5 * cosv * rn
        uy = 1.5 * sinv * rn
        uz = 0.15 * rn
        a0x = base[:, 0:1] + ux[:, 0:1]
        a0y = base[:, 1:2] + uy[:, 0:1]
        a0z = base[:, 2:3] + uz[:, 0:1]
        a1x = a0x + ux[:, 1:2]
        a1y = a0y + uy[:, 1:2]
        a1z = a0z + uz[:, 1:2]
        a2x = a1x + ux[:, 2:3]
        a2y = a1y + uy[:, 2:3]
        a2z = a1z + uz[:, 2:3]
        prow = jnp.concatenate(
            [a0x, a0y, a0z, a1x, a1y, a1z, a2x, a2y, a2z], axis=1)

        poss_ref[t] = prow
        angs_ref[t] = angle
        frow = jnp.concatenate([h_new, dt, angle], axis=1)
        v_ref[t] = _dot(frow, wvf_ref[...])
        h1_ref[t] = _dot(frow, w1f_ref[...]) + b1
        return h_new, ctx_new

    h0 = jnp.zeros((_B, _H), f32)
    c0 = jnp.zeros((_B, _C), f32)
    lax.fori_loop(0, lmax_ref[0], body, (h0, c0))


def kernel(inputs, structure, W_ih, b_ih, W_hh, b_hh, Wa, ba, W1, b1, W2, Wv):
    f32 = jnp.float32
    counts = jnp.bincount(structure, length=_B)
    starts = jnp.concatenate(
        [jnp.zeros((1,), counts.dtype), jnp.cumsum(counts)[:-1]])
    pos = jnp.arange(_TOTAL) - starts[structure]
    data = jnp.zeros((_L, _B, _IN), f32).at[pos, structure].set(inputs)
    lmax = jnp.max(counts).astype(jnp.int32).reshape(1)

    WihT = W_ih.T.astype(f32)
    WhhT = W_hh.T.astype(f32)
    mw = jnp.sum(jnp.abs(W2), axis=0).reshape(1, _HEADS)

    poss_out, angs_out = pl.pallas_call(
        _rnn_kernel,
        in_specs=[pl.BlockSpec(memory_space=pltpu.SMEM)] +
                 [pl.BlockSpec(memory_space=pltpu.VMEM)] * 18,
        out_specs=[pl.BlockSpec(memory_space=pltpu.VMEM)] * 2,
        out_shape=[
            jax.ShapeDtypeStruct((_L, _B, 9), f32),
            jax.ShapeDtypeStruct((_L, _B, 3), f32),
        ],
        scratch_shapes=[
            pltpu.VMEM((_L, _B, _C), f32),
            pltpu.VMEM((_L, _B, _A), f32),
        ],
    )(
        lmax, data,
        WihT[:, 0:_H], WihT[:, _H:2 * _H], WihT[:, 2 * _H:3 * _H],
        b_ih.reshape(1, 3 * _H),
        WhhT[:, 0:_H], WhhT[:, _H:2 * _H], WhhT[:, 2 * _H:3 * _H],
        b_hh.reshape(1, 3 * _H),
        Wa, ba.reshape(1, 3),
        W1[0:_H + _IN + 3], b1.reshape(1, _A), W1[_H + _IN + 3:],
        W2, mw, Wv[0:_H + _IN + 3], Wv[_H + _IN + 3:],
    )

    pos_flat = poss_out[pos, structure].reshape(_TOTAL, 3, 3)
    ang_flat = angs_out[pos, structure]
    return pos_flat, ang_flat
